# Initial kernel scaffold; baseline (speedup 1.0000x reference)
import jax
import jax.numpy as jnp
from jax.experimental import pallas as pl


def kernel(x, edge_index, edge_attr, Wm, bm, Wih, Whh, bih, bhh):
    raise NotImplementedError("write your pallas kernel here")


# speedup vs baseline: 3.0144x; 3.0144x over previous
import functools

import jax
import jax.numpy as jnp
from jax import lax
from jax.experimental import pallas as pl
from jax.experimental.pallas import tpu as pltpu
from jax.experimental.pallas import tpu_sc as plsc

N = 10000
E = 320000
ND = 128
ED = 16
NC = 2
NS = 16
NW = NC * NS
EPW = E // NW
K = 80
CH = EPW // K
RA = 624

_sc_mesh = plsc.VectorSubcoreMesh(core_axis_name="c", subcore_axis_name="s")


@functools.partial(
    pl.kernel,
    out_type=[jax.ShapeDtypeStruct((NC, N, ND), jnp.float32)],
    mesh=_sc_mesh,
    scratch_types=[
        pltpu.VMEM_SHARED((N, ND), jnp.float32),
        pltpu.VMEM((K,), jnp.int32),
        pltpu.VMEM((K,), jnp.int32),
        pltpu.VMEM((K, ND), jnp.float32),
        pltpu.SemaphoreType.DMA,
    ],
)
def _sc_seg(h, src1d, dst1d, z128,
            s_out,
            s_sh, src_vec, dst_vec, rows, sem):
    c = lax.axis_index("c")
    s = lax.axis_index("s")
    wid = c * NS + s
    pltpu.sync_copy(z128, s_sh.at[pl.ds(s * RA, RA)])

    @pl.when(s == NS - 1)
    def _():
        pltpu.sync_copy(z128.at[pl.ds(0, N - NS * RA)],
                        s_sh.at[pl.ds(NS * RA, N - NS * RA)])

    plsc.subcore_barrier()

    def body(j, carry):
        base = wid * EPW + j * K
        pltpu.sync_copy(src1d.at[pl.ds(base, K)], src_vec)
        pltpu.sync_copy(dst1d.at[pl.ds(base, K)], dst_vec)
        pltpu.async_copy(h.at[src_vec], rows, sem).wait()
        pltpu.sync_copy(rows, s_sh.at[dst_vec], add=True)
        return carry

    lax.fori_loop(0, CH, body, 0)
    plsc.subcore_barrier()
    pltpu.sync_copy(s_sh.at[pl.ds(s * RA, RA)], s_out.at[c, pl.ds(s * RA, RA)])

    @pl.when(s == NS - 1)
    def _():
        pltpu.sync_copy(s_sh.at[pl.ds(NS * RA, N - NS * RA)],
                        s_out.at[c, pl.ds(NS * RA, N - NS * RA)])


@functools.partial(
    pl.kernel,
    out_type=[jax.ShapeDtypeStruct((NC, N, ND), jnp.float32)],
    mesh=_sc_mesh,
    scratch_types=[
        pltpu.VMEM_SHARED((N, ND), jnp.float32),
        pltpu.VMEM((K,), jnp.int32),
        pltpu.VMEM((K,), jnp.int32),
        pltpu.VMEM((K, ND), jnp.float32),
        pltpu.SemaphoreType.DMA,
    ],
)
def _sc_td(ea_pad, perm1d, dst1d, z128,
           td_out,
           td_sh, perm_vec, dst_vec, erows, sem):
    c = lax.axis_index("c")
    s = lax.axis_index("s")
    wid = c * NS + s
    pltpu.sync_copy(z128, td_sh.at[pl.ds(s * RA, RA)])

    @pl.when(s == NS - 1)
    def _():
        pltpu.sync_copy(z128.at[pl.ds(0, N - NS * RA)],
                        td_sh.at[pl.ds(NS * RA, N - NS * RA)])

    plsc.subcore_barrier()

    def body(j, carry):
        base = wid * EPW + j * K
        pltpu.sync_copy(perm1d.at[pl.ds(base, K)], perm_vec)
        pltpu.sync_copy(dst1d.at[pl.ds(base, K)], dst_vec)
        pltpu.async_copy(ea_pad.at[perm_vec], erows, sem).wait()
        pltpu.sync_copy(erows, td_sh.at[dst_vec], add=True)
        return carry

    lax.fori_loop(0, CH, body, 0)
    plsc.subcore_barrier()
    pltpu.sync_copy(td_sh.at[pl.ds(s * RA, RA)], td_out.at[c, pl.ds(s * RA, RA)])

    @pl.when(s == NS - 1)
    def _():
        pltpu.sync_copy(td_sh.at[pl.ds(NS * RA, N - NS * RA)],
                        td_out.at[c, pl.ds(NS * RA, N - NS * RA)])


BLK = 1000


def _tc_body(h_ref, sp_ref, tdp_ref, wst_ref, wdt_ref, wet_ref,
             bm_ref, wiht_ref, whht_ref, bih_ref, bhh_ref, out_ref):
    h = h_ref[...]
    hbf = h.astype(jnp.bfloat16).astype(jnp.float32)
    s = sp_ref[0] + sp_ref[1]
    td = tdp_ref[0] + tdp_ref[1]
    t = td[:, :ED]
    deg = td[:, ED:ED + 1]
    dot = functools.partial(jnp.dot, preferred_element_type=jnp.float32,
                            precision=jax.lax.Precision.HIGHEST)
    a = (dot(s, wst_ref[...]) + dot(deg * hbf, wdt_ref[...])
         + dot(t, wet_ref[...]) + deg * bm_ref[...])
    abf = a.astype(jnp.bfloat16).astype(jnp.float32)
    gi = dot(abf, wiht_ref[...]) + bih_ref[...]
    gh = dot(hbf, whht_ref[...]) + bhh_ref[...]
    r = jax.nn.sigmoid(gi[:, :ND] + gh[:, :ND])
    z = jax.nn.sigmoid(gi[:, ND:2 * ND] + gh[:, ND:2 * ND])
    n = jnp.tanh(gi[:, 2 * ND:] + r * gh[:, 2 * ND:])
    out_ref[...] = (1.0 - z) * n + z * h


def _tc_round(h, sp, tdp, wst, wdt, wet, bm_row, wiht, whht, bih_row, bhh_row):
    grid = (N // BLK,)
    row_blk = lambda i: (i, 0)
    full = lambda shape: pl.BlockSpec(shape, lambda i: (0,) * len(shape))
    return pl.pallas_call(
        _tc_body,
        grid=grid,
        in_specs=[
            pl.BlockSpec((BLK, ND), row_blk),
            pl.BlockSpec((NC, BLK, ND), lambda i: (0, i, 0)),
            pl.BlockSpec((NC, BLK, ND), lambda i: (0, i, 0)),
            full((ND, ND)),
            full((ND, ND)),
            full((ED, ND)),
            full((1, ND)),
            full((ND, 3 * ND)),
            full((ND, 3 * ND)),
            full((1, 3 * ND)),
            full((1, 3 * ND)),
        ],
        out_specs=pl.BlockSpec((BLK, ND), row_blk),
        out_shape=jax.ShapeDtypeStruct((N, ND), jnp.float32),
    )(h, sp, tdp, wst, wdt, wet, bm_row, wiht, whht, bih_row, bhh_row)


def kernel(x, edge_index, edge_attr, Wm, bm, Wih, Whh, bih, bhh):
    perm = jnp.argsort(edge_index[1])
    perm_s = perm.reshape(K, E // K).T.reshape(E)
    src_s = edge_index[0][perm_s]
    dst_s = edge_index[1][perm_s]
    z128 = jnp.zeros((RA, ND), jnp.float32)

    def bf(v):
        return v.astype(jnp.bfloat16).astype(jnp.float32)

    ea_pad = jnp.concatenate(
        [bf(edge_attr),
         jnp.ones((E, 1), jnp.float32),
         jnp.zeros((E, ND - ED - 1), jnp.float32)], axis=1)

    (tdp,) = _sc_td(ea_pad, perm_s, dst_s, z128)
    (sp,) = _sc_seg(bf(x), src_s, dst_s, z128)

    h = x
    for t in range(2):
        if t == 1:
            (sp,) = _sc_seg(bf(h), src_s, dst_s, z128)
        wst = bf(Wm[t][:, :ND].T)
        wdt = bf(Wm[t][:, ND:2 * ND].T)
        wet = bf(Wm[t][:, 2 * ND:].T)
        h = _tc_round(h, sp, tdp, wst, wdt, wet, bm[t][None],
                      bf(Wih[t].T), bf(Whh[t].T), bih[t][None], bhh[t][None])
    return h


# speedup vs baseline: 4.3082x; 1.4292x over previous
import functools

import jax
import jax.numpy as jnp
from jax import lax
from jax.experimental import pallas as pl
from jax.experimental.pallas import tpu as pltpu
from jax.experimental.pallas import tpu_sc as plsc

N = 10000
E = 320000
ND = 128
ED = 16
NC = 2
NS = 16
NW = NC * NS
EPW = E // NW
K = 80
CH = EPW // K
RA = 624

_sc_mesh = plsc.VectorSubcoreMesh(core_axis_name="c", subcore_axis_name="s")


@functools.partial(
    pl.kernel,
    out_type=[jax.ShapeDtypeStruct((NC, N, ND), jnp.float32)],
    mesh=_sc_mesh,
    scratch_types=[
        pltpu.VMEM_SHARED((N, ND), jnp.float32),
        pltpu.VMEM((K,), jnp.int32),
        pltpu.VMEM((K,), jnp.int32),
        pltpu.VMEM((K, ND), jnp.float32),
        pltpu.SemaphoreType.DMA,
    ],
)
def _sc_seg(h, src1d, dst1d, z128,
            s_out,
            s_sh, src_vec, dst_vec, rows, sem):
    c = lax.axis_index("c")
    s = lax.axis_index("s")
    wid = c * NS + s
    pltpu.sync_copy(z128, s_sh.at[pl.ds(s * RA, RA)])

    @pl.when(s == NS - 1)
    def _():
        pltpu.sync_copy(z128.at[pl.ds(0, N - NS * RA)],
                        s_sh.at[pl.ds(NS * RA, N - NS * RA)])

    plsc.subcore_barrier()

    def body(j, carry):
        base = wid * EPW + j * K
        pltpu.sync_copy(src1d.at[pl.ds(base, K)], src_vec)
        pltpu.sync_copy(dst1d.at[pl.ds(base, K)], dst_vec)
        pltpu.async_copy(h.at[src_vec], rows, sem).wait()
        pltpu.sync_copy(rows, s_sh.at[dst_vec], add=True)
        return carry

    lax.fori_loop(0, CH, body, 0)
    plsc.subcore_barrier()
    pltpu.sync_copy(s_sh.at[pl.ds(s * RA, RA)], s_out.at[c, pl.ds(s * RA, RA)])

    @pl.when(s == NS - 1)
    def _():
        pltpu.sync_copy(s_sh.at[pl.ds(NS * RA, N - NS * RA)],
                        s_out.at[c, pl.ds(NS * RA, N - NS * RA)])


@functools.partial(
    pl.kernel,
    out_type=[jax.ShapeDtypeStruct((NC, N, ND), jnp.float32)],
    mesh=_sc_mesh,
    scratch_types=[
        pltpu.VMEM_SHARED((N, ND), jnp.float32),
        pltpu.VMEM((K,), jnp.int32),
        pltpu.VMEM((K, ND), jnp.float32),
        pltpu.SemaphoreType.DMA,
    ],
)
def _sc_td(ea_pad, dst1d, z128,
           td_out,
           td_sh, dst_vec, erows, sem):
    c = lax.axis_index("c")
    s = lax.axis_index("s")
    wid = c * NS + s
    pltpu.sync_copy(z128, td_sh.at[pl.ds(s * RA, RA)])

    @pl.when(s == NS - 1)
    def _():
        pltpu.sync_copy(z128.at[pl.ds(0, N - NS * RA)],
                        td_sh.at[pl.ds(NS * RA, N - NS * RA)])

    plsc.subcore_barrier()

    def body(j, carry):
        base = wid * EPW + j * K
        pltpu.sync_copy(dst1d.at[pl.ds(base, K)], dst_vec)
        pltpu.sync_copy(ea_pad.at[pl.ds(base, K)], erows)
        pltpu.sync_copy(erows, td_sh.at[dst_vec], add=True)
        return carry

    lax.fori_loop(0, CH, body, 0)
    plsc.subcore_barrier()
    pltpu.sync_copy(td_sh.at[pl.ds(s * RA, RA)], td_out.at[c, pl.ds(s * RA, RA)])

    @pl.when(s == NS - 1)
    def _():
        pltpu.sync_copy(td_sh.at[pl.ds(NS * RA, N - NS * RA)],
                        td_out.at[c, pl.ds(NS * RA, N - NS * RA)])


BLK = 1000


def _tc_body(h_ref, sp_ref, tdp_ref, wst_ref, wdt_ref, wet_ref,
             bm_ref, wiht_ref, whht_ref, bih_ref, bhh_ref, out_ref):
    h = h_ref[...]
    hbf = h.astype(jnp.bfloat16).astype(jnp.float32)
    s = sp_ref[0] + sp_ref[1]
    td = tdp_ref[0] + tdp_ref[1]
    t = td[:, :ED]
    deg = td[:, ED:ED + 1]
    dot = functools.partial(jnp.dot, preferred_element_type=jnp.float32,
                            precision=jax.lax.Precision.HIGHEST)
    a = (dot(s, wst_ref[...]) + dot(deg * hbf, wdt_ref[...])
         + dot(t, wet_ref[...]) + deg * bm_ref[...])
    abf = a.astype(jnp.bfloat16).astype(jnp.float32)
    gi = dot(abf, wiht_ref[...]) + bih_ref[...]
    gh = dot(hbf, whht_ref[...]) + bhh_ref[...]
    r = jax.nn.sigmoid(gi[:, :ND] + gh[:, :ND])
    z = jax.nn.sigmoid(gi[:, ND:2 * ND] + gh[:, ND:2 * ND])
    n = jnp.tanh(gi[:, 2 * ND:] + r * gh[:, 2 * ND:])
    out_ref[...] = (1.0 - z) * n + z * h


def _tc_round(h, sp, tdp, wst, wdt, wet, bm_row, wiht, whht, bih_row, bhh_row):
    grid = (N // BLK,)
    row_blk = lambda i: (i, 0)
    full = lambda shape: pl.BlockSpec(shape, lambda i: (0,) * len(shape))
    return pl.pallas_call(
        _tc_body,
        grid=grid,
        in_specs=[
            pl.BlockSpec((BLK, ND), row_blk),
            pl.BlockSpec((NC, BLK, ND), lambda i: (0, i, 0)),
            pl.BlockSpec((NC, BLK, ND), lambda i: (0, i, 0)),
            full((ND, ND)),
            full((ND, ND)),
            full((ED, ND)),
            full((1, ND)),
            full((ND, 3 * ND)),
            full((ND, 3 * ND)),
            full((1, 3 * ND)),
            full((1, 3 * ND)),
        ],
        out_specs=pl.BlockSpec((BLK, ND), row_blk),
        out_shape=jax.ShapeDtypeStruct((N, ND), jnp.float32),
    )(h, sp, tdp, wst, wdt, wet, bm_row, wiht, whht, bih_row, bhh_row)


def kernel(x, edge_index, edge_attr, Wm, bm, Wih, Whh, bih, bhh):
    src_s = edge_index[0]
    dst_s = edge_index[1]
    z128 = jnp.zeros((RA, ND), jnp.float32)

    def bf(v):
        return v.astype(jnp.bfloat16).astype(jnp.float32)

    ea_pad = jnp.concatenate(
        [bf(edge_attr),
         jnp.ones((E, 1), jnp.float32),
         jnp.zeros((E, ND - ED - 1), jnp.float32)], axis=1)

    (tdp,) = _sc_td(ea_pad, dst_s, z128)
    (sp,) = _sc_seg(bf(x), src_s, dst_s, z128)

    h = x
    for t in range(2):
        if t == 1:
            (sp,) = _sc_seg(bf(h), src_s, dst_s, z128)
        wst = bf(Wm[t][:, :ND].T)
        wdt = bf(Wm[t][:, ND:2 * ND].T)
        wet = bf(Wm[t][:, 2 * ND:].T)
        h = _tc_round(h, sp, tdp, wst, wdt, wet, bm[t][None],
                      bf(Wih[t].T), bf(Whh[t].T), bih[t][None], bhh[t][None])
    return h


# speedup vs baseline: 5.5360x; 1.2850x over previous
import functools

import jax
import jax.numpy as jnp
from jax import lax
from jax.experimental import pallas as pl
from jax.experimental.pallas import tpu as pltpu
from jax.experimental.pallas import tpu_sc as plsc

N = 10000
E = 320000
ND = 128
ED = 16
NC = 2
NS = 16
NW = NC * NS
EPW = E // NW
K = 80
CH = EPW // K
RA = 624

_sc_mesh = plsc.VectorSubcoreMesh(core_axis_name="c", subcore_axis_name="s")


@functools.partial(
    pl.kernel,
    out_type=[jax.ShapeDtypeStruct((NC, N, ND), jnp.float32)],
    mesh=_sc_mesh,
    scratch_types=[
        pltpu.VMEM_SHARED((N, ND), jnp.float32),
        pltpu.VMEM((K,), jnp.int32),
        pltpu.VMEM((K,), jnp.int32),
        pltpu.VMEM((K,), jnp.int32),
        pltpu.VMEM((K,), jnp.int32),
        pltpu.VMEM((K, ND), jnp.float32),
        pltpu.VMEM((K, ND), jnp.float32),
        pltpu.SemaphoreType.DMA,
        pltpu.SemaphoreType.DMA,
    ],
)
def _sc_seg(h, src1d, dst1d, z128,
            s_out,
            s_sh, src_a, dst_a, src_b, dst_b, rows_a, rows_b, sem_a, sem_b):
    c = lax.axis_index("c")
    s = lax.axis_index("s")
    wid = c * NS + s
    pltpu.sync_copy(z128, s_sh.at[pl.ds(s * RA, RA)])

    @pl.when(s == NS - 1)
    def _():
        pltpu.sync_copy(z128.at[pl.ds(0, N - NS * RA)],
                        s_sh.at[pl.ds(NS * RA, N - NS * RA)])

    plsc.subcore_barrier()

    def load_idx(j, sv, dv):
        base = wid * EPW + j * K
        pltpu.sync_copy(src1d.at[pl.ds(base, K)], sv)
        pltpu.sync_copy(dst1d.at[pl.ds(base, K)], dv)

    load_idx(0, src_a, dst_a)
    pltpu.async_copy(h.at[src_a], rows_a, sem_a)

    def body(i, carry):
        a = 2 * i
        load_idx(a + 1, src_b, dst_b)
        pltpu.make_async_copy(h.at[src_a], rows_a, sem_a).wait()
        pltpu.async_copy(h.at[src_b], rows_b, sem_b)
        pltpu.sync_copy(rows_a, s_sh.at[dst_a], add=True)
        load_idx(a + 2, src_a, dst_a)
        pltpu.make_async_copy(h.at[src_b], rows_b, sem_b).wait()
        pltpu.async_copy(h.at[src_a], rows_a, sem_a)
        pltpu.sync_copy(rows_b, s_sh.at[dst_b], add=True)
        return carry

    lax.fori_loop(0, (CH - 1) // 2, body, 0)
    pltpu.make_async_copy(h.at[src_a], rows_a, sem_a).wait()
    pltpu.sync_copy(rows_a, s_sh.at[dst_a], add=True)
    plsc.subcore_barrier()
    pltpu.sync_copy(s_sh.at[pl.ds(s * RA, RA)], s_out.at[c, pl.ds(s * RA, RA)])

    @pl.when(s == NS - 1)
    def _():
        pltpu.sync_copy(s_sh.at[pl.ds(NS * RA, N - NS * RA)],
                        s_out.at[c, pl.ds(NS * RA, N - NS * RA)])


@functools.partial(
    pl.kernel,
    out_type=[jax.ShapeDtypeStruct((NC, N, ND), jnp.float32)],
    mesh=_sc_mesh,
    scratch_types=[
        pltpu.VMEM_SHARED((N, ND), jnp.float32),
        pltpu.VMEM((K,), jnp.int32),
        pltpu.VMEM((K, ND), jnp.float32),
        pltpu.SemaphoreType.DMA,
    ],
)
def _sc_td(ea_pad, dst1d, z128,
           td_out,
           td_sh, dst_vec, erows, sem):
    c = lax.axis_index("c")
    s = lax.axis_index("s")
    wid = c * NS + s
    pltpu.sync_copy(z128, td_sh.at[pl.ds(s * RA, RA)])

    @pl.when(s == NS - 1)
    def _():
        pltpu.sync_copy(z128.at[pl.ds(0, N - NS * RA)],
                        td_sh.at[pl.ds(NS * RA, N - NS * RA)])

    plsc.subcore_barrier()

    def body(j, carry):
        base = wid * EPW + j * K
        pltpu.sync_copy(dst1d.at[pl.ds(base, K)], dst_vec)
        pltpu.sync_copy(ea_pad.at[pl.ds(base, K)], erows)
        pltpu.sync_copy(erows, td_sh.at[dst_vec], add=True)
        return carry

    lax.fori_loop(0, CH, body, 0)
    plsc.subcore_barrier()
    pltpu.sync_copy(td_sh.at[pl.ds(s * RA, RA)], td_out.at[c, pl.ds(s * RA, RA)])

    @pl.when(s == NS - 1)
    def _():
        pltpu.sync_copy(td_sh.at[pl.ds(NS * RA, N - NS * RA)],
                        td_out.at[c, pl.ds(NS * RA, N - NS * RA)])


BLK = 1000


def _tc_body(h_ref, sp_ref, tdp_ref, wst_ref, wdt_ref, wet_ref,
             bm_ref, wiht_ref, whht_ref, bih_ref, bhh_ref, out_ref):
    h = h_ref[...]
    hbf = h.astype(jnp.bfloat16).astype(jnp.float32)
    s = sp_ref[0] + sp_ref[1]
    td = tdp_ref[0] + tdp_ref[1]
    t = td[:, :ED]
    deg = td[:, ED:ED + 1]
    dot = functools.partial(jnp.dot, preferred_element_type=jnp.float32,
                            precision=jax.lax.Precision.HIGHEST)
    a = (dot(s, wst_ref[...]) + dot(deg * hbf, wdt_ref[...])
         + dot(t, wet_ref[...]) + deg * bm_ref[...])
    abf = a.astype(jnp.bfloat16).astype(jnp.float32)
    gi = dot(abf, wiht_ref[...]) + bih_ref[...]
    gh = dot(hbf, whht_ref[...]) + bhh_ref[...]
    r = jax.nn.sigmoid(gi[:, :ND] + gh[:, :ND])
    z = jax.nn.sigmoid(gi[:, ND:2 * ND] + gh[:, ND:2 * ND])
    n = jnp.tanh(gi[:, 2 * ND:] + r * gh[:, 2 * ND:])
    out_ref[...] = (1.0 - z) * n + z * h


def _tc_round(h, sp, tdp, wst, wdt, wet, bm_row, wiht, whht, bih_row, bhh_row):
    grid = (N // BLK,)
    row_blk = lambda i: (i, 0)
    full = lambda shape: pl.BlockSpec(shape, lambda i: (0,) * len(shape))
    return pl.pallas_call(
        _tc_body,
        grid=grid,
        in_specs=[
            pl.BlockSpec((BLK, ND), row_blk),
            pl.BlockSpec((NC, BLK, ND), lambda i: (0, i, 0)),
            pl.BlockSpec((NC, BLK, ND), lambda i: (0, i, 0)),
            full((ND, ND)),
            full((ND, ND)),
            full((ED, ND)),
            full((1, ND)),
            full((ND, 3 * ND)),
            full((ND, 3 * ND)),
            full((1, 3 * ND)),
            full((1, 3 * ND)),
        ],
        out_specs=pl.BlockSpec((BLK, ND), row_blk),
        out_shape=jax.ShapeDtypeStruct((N, ND), jnp.float32),
    )(h, sp, tdp, wst, wdt, wet, bm_row, wiht, whht, bih_row, bhh_row)


def kernel(x, edge_index, edge_attr, Wm, bm, Wih, Whh, bih, bhh):
    src_s = edge_index[0]
    dst_s = edge_index[1]
    z128 = jnp.zeros((RA, ND), jnp.float32)

    def bf(v):
        return v.astype(jnp.bfloat16).astype(jnp.float32)

    ea_pad = jnp.concatenate(
        [bf(edge_attr),
         jnp.ones((E, 1), jnp.float32),
         jnp.zeros((E, ND - ED - 1), jnp.float32)], axis=1)

    (tdp,) = _sc_td(ea_pad, dst_s, z128)
    (sp,) = _sc_seg(bf(x), src_s, dst_s, z128)

    h = x
    for t in range(2):
        if t == 1:
            (sp,) = _sc_seg(bf(h), src_s, dst_s, z128)
        wst = bf(Wm[t][:, :ND].T)
        wdt = bf(Wm[t][:, ND:2 * ND].T)
        wet = bf(Wm[t][:, 2 * ND:].T)
        h = _tc_round(h, sp, tdp, wst, wdt, wet, bm[t][None],
                      bf(Wih[t].T), bf(Whh[t].T), bih[t][None], bhh[t][None])
    return h


# speedup vs baseline: 6.3506x; 1.1472x over previous
import functools

import jax
import jax.numpy as jnp
from jax import lax
from jax.experimental import pallas as pl
from jax.experimental.pallas import tpu as pltpu
from jax.experimental.pallas import tpu_sc as plsc

N = 10000
E = 320000
ND = 128
ED = 16
NC = 2
NS = 16
NW = NC * NS
EPW = E // NW
K = 80
CH = EPW // K
RA = 624

_sc_mesh = plsc.VectorSubcoreMesh(core_axis_name="c", subcore_axis_name="s")


@functools.partial(
    pl.kernel,
    out_type=[jax.ShapeDtypeStruct((NC, N, ND), jnp.float32)],
    mesh=_sc_mesh,
    scratch_types=[
        pltpu.VMEM_SHARED((N, ND), jnp.float32),
        pltpu.VMEM((K,), jnp.int32),
        pltpu.VMEM((K,), jnp.int32),
        pltpu.VMEM((K,), jnp.int32),
        pltpu.VMEM((K,), jnp.int32),
        pltpu.VMEM((K, ND), jnp.float32),
        pltpu.VMEM((K, ND), jnp.float32),
        pltpu.SemaphoreType.DMA,
        pltpu.SemaphoreType.DMA,
    ],
)
def _sc_seg(h, src1d, dst1d, z128,
            s_out,
            s_sh, src_a, dst_a, src_b, dst_b, rows_a, rows_b, sem_a, sem_b):
    c = lax.axis_index("c")
    s = lax.axis_index("s")
    wid = c * NS + s
    pltpu.sync_copy(z128, s_sh.at[pl.ds(s * RA, RA)])

    @pl.when(s == NS - 1)
    def _():
        pltpu.sync_copy(z128.at[pl.ds(0, N - NS * RA)],
                        s_sh.at[pl.ds(NS * RA, N - NS * RA)])

    plsc.subcore_barrier()

    def load_idx(j, sv, dv):
        base = wid * EPW + j * K
        pltpu.sync_copy(src1d.at[pl.ds(base, K)], sv)
        pltpu.sync_copy(dst1d.at[pl.ds(base, K)], dv)

    load_idx(0, src_a, dst_a)
    pltpu.async_copy(h.at[src_a], rows_a, sem_a)

    def body(i, carry):
        a = 2 * i
        load_idx(a + 1, src_b, dst_b)
        pltpu.make_async_copy(h.at[src_a], rows_a, sem_a).wait()
        pltpu.async_copy(h.at[src_b], rows_b, sem_b)
        pltpu.sync_copy(rows_a, s_sh.at[dst_a], add=True)
        load_idx(a + 2, src_a, dst_a)
        pltpu.make_async_copy(h.at[src_b], rows_b, sem_b).wait()
        pltpu.async_copy(h.at[src_a], rows_a, sem_a)
        pltpu.sync_copy(rows_b, s_sh.at[dst_b], add=True)
        return carry

    lax.fori_loop(0, (CH - 1) // 2, body, 0)
    pltpu.make_async_copy(h.at[src_a], rows_a, sem_a).wait()
    pltpu.sync_copy(rows_a, s_sh.at[dst_a], add=True)
    plsc.subcore_barrier()
    pltpu.sync_copy(s_sh.at[pl.ds(s * RA, RA)], s_out.at[c, pl.ds(s * RA, RA)])

    @pl.when(s == NS - 1)
    def _():
        pltpu.sync_copy(s_sh.at[pl.ds(NS * RA, N - NS * RA)],
                        s_out.at[c, pl.ds(NS * RA, N - NS * RA)])


@functools.partial(
    pl.kernel,
    out_type=[jax.ShapeDtypeStruct((NC, N, ND), jnp.float32)],
    mesh=_sc_mesh,
    scratch_types=[
        pltpu.VMEM_SHARED((N, ND), jnp.float32),
        pltpu.VMEM((K,), jnp.int32),
        pltpu.VMEM((K,), jnp.int32),
        pltpu.VMEM((K, ND), jnp.float32),
        pltpu.VMEM((K, ND), jnp.float32),
        pltpu.SemaphoreType.DMA,
        pltpu.SemaphoreType.DMA,
    ],
)
def _sc_td(ea_pad, dst1d, z128,
           td_out,
           td_sh, dst_a, dst_b, erows_a, erows_b, sem_a, sem_b):
    c = lax.axis_index("c")
    s = lax.axis_index("s")
    wid = c * NS + s
    pltpu.sync_copy(z128, td_sh.at[pl.ds(s * RA, RA)])

    @pl.when(s == NS - 1)
    def _():
        pltpu.sync_copy(z128.at[pl.ds(0, N - NS * RA)],
                        td_sh.at[pl.ds(NS * RA, N - NS * RA)])

    plsc.subcore_barrier()

    def stage(j, dv, er, sem):
        base = wid * EPW + j * K
        pltpu.sync_copy(dst1d.at[pl.ds(base, K)], dv)
        return pltpu.async_copy(ea_pad.at[pl.ds(base, K)], er, sem)

    def wait(j, er, sem):
        base = wid * EPW + j * K
        pltpu.make_async_copy(ea_pad.at[pl.ds(base, K)], er, sem).wait()

    stage(0, dst_a, erows_a, sem_a)

    def body(i, carry):
        a = 2 * i
        stage(a + 1, dst_b, erows_b, sem_b)
        wait(a, erows_a, sem_a)
        pltpu.sync_copy(erows_a, td_sh.at[dst_a], add=True)
        stage(a + 2, dst_a, erows_a, sem_a)
        wait(a + 1, erows_b, sem_b)
        pltpu.sync_copy(erows_b, td_sh.at[dst_b], add=True)
        return carry

    lax.fori_loop(0, (CH - 1) // 2, body, 0)
    wait(CH - 1, erows_a, sem_a)
    pltpu.sync_copy(erows_a, td_sh.at[dst_a], add=True)
    plsc.subcore_barrier()
    pltpu.sync_copy(td_sh.at[pl.ds(s * RA, RA)], td_out.at[c, pl.ds(s * RA, RA)])

    @pl.when(s == NS - 1)
    def _():
        pltpu.sync_copy(td_sh.at[pl.ds(NS * RA, N - NS * RA)],
                        td_out.at[c, pl.ds(NS * RA, N - NS * RA)])


BLK = 1000


def _tc_body(h_ref, sp_ref, tdp_ref, wst_ref, wdt_ref, wet_ref,
             bm_ref, wiht_ref, whht_ref, bih_ref, bhh_ref, out_ref):
    h = h_ref[...]
    hbf = h.astype(jnp.bfloat16).astype(jnp.float32)
    s = sp_ref[0] + sp_ref[1]
    td = tdp_ref[0] + tdp_ref[1]
    t = td[:, :ED]
    deg = td[:, ED:ED + 1]
    dot = functools.partial(jnp.dot, preferred_element_type=jnp.float32,
                            precision=jax.lax.Precision.HIGHEST)
    a = (dot(s, wst_ref[...]) + dot(deg * hbf, wdt_ref[...])
         + dot(t, wet_ref[...]) + deg * bm_ref[...])
    abf = a.astype(jnp.bfloat16).astype(jnp.float32)
    gi = dot(abf, wiht_ref[...]) + bih_ref[...]
    gh = dot(hbf, whht_ref[...]) + bhh_ref[...]
    r = jax.nn.sigmoid(gi[:, :ND] + gh[:, :ND])
    z = jax.nn.sigmoid(gi[:, ND:2 * ND] + gh[:, ND:2 * ND])
    n = jnp.tanh(gi[:, 2 * ND:] + r * gh[:, 2 * ND:])
    out_ref[...] = (1.0 - z) * n + z * h


def _tc_round(h, sp, tdp, wst, wdt, wet, bm_row, wiht, whht, bih_row, bhh_row):
    grid = (N // BLK,)
    row_blk = lambda i: (i, 0)
    full = lambda shape: pl.BlockSpec(shape, lambda i: (0,) * len(shape))
    return pl.pallas_call(
        _tc_body,
        grid=grid,
        in_specs=[
            pl.BlockSpec((BLK, ND), row_blk),
            pl.BlockSpec((NC, BLK, ND), lambda i: (0, i, 0)),
            pl.BlockSpec((NC, BLK, ND), lambda i: (0, i, 0)),
            full((ND, ND)),
            full((ND, ND)),
            full((ED, ND)),
            full((1, ND)),
            full((ND, 3 * ND)),
            full((ND, 3 * ND)),
            full((1, 3 * ND)),
            full((1, 3 * ND)),
        ],
        out_specs=pl.BlockSpec((BLK, ND), row_blk),
        out_shape=jax.ShapeDtypeStruct((N, ND), jnp.float32),
    )(h, sp, tdp, wst, wdt, wet, bm_row, wiht, whht, bih_row, bhh_row)


def kernel(x, edge_index, edge_attr, Wm, bm, Wih, Whh, bih, bhh):
    src_s = edge_index[0]
    dst_s = edge_index[1]
    z128 = jnp.zeros((RA, ND), jnp.float32)

    def bf(v):
        return v.astype(jnp.bfloat16).astype(jnp.float32)

    ea_pad = jnp.concatenate(
        [bf(edge_attr),
         jnp.ones((E, 1), jnp.float32),
         jnp.zeros((E, ND - ED - 1), jnp.float32)], axis=1)

    (tdp,) = _sc_td(ea_pad, dst_s, z128)
    (sp,) = _sc_seg(bf(x), src_s, dst_s, z128)

    h = x
    for t in range(2):
        if t == 1:
            (sp,) = _sc_seg(bf(h), src_s, dst_s, z128)
        wst = bf(Wm[t][:, :ND].T)
        wdt = bf(Wm[t][:, ND:2 * ND].T)
        wet = bf(Wm[t][:, 2 * ND:].T)
        h = _tc_round(h, sp, tdp, wst, wdt, wet, bm[t][None],
                      bf(Wih[t].T), bf(Whh[t].T), bih[t][None], bhh[t][None])
    return h
